# baseline (speedup 1.0000x reference)
import functools

import jax
import jax.numpy as jnp
from jax import lax
from jax.experimental import pallas as pl
from jax.experimental.pallas import tpu as pltpu
from jax.experimental.pallas import tpu_sc as plsc

N_NODES = 10000
N_EDGES = 320000
D_FEAT = 128
D_EDGE = 16
D_GLOBAL = 16
D_OUT = D_EDGE + 2 * D_FEAT + D_GLOBAL
D_PAD = 384

_NC = 2
_NS = 16
_NW = _NC * _NS
_E_PER_W = N_EDGES // _NW
_B = 88
_STEPS = _E_PER_W // _B

_ROWS_MAIN = 624
_ROWS_LAST = N_NODES - 15 * _ROWS_MAIN
_BLD = 8

_mesh = plsc.VectorSubcoreMesh(core_axis_name="c", subcore_axis_name="s")


@functools.partial(
    pl.kernel,
    out_type=jax.ShapeDtypeStruct((N_EDGES, D_PAD), jnp.float32),
    mesh=_mesh,
    scratch_types=[
        pltpu.VMEM_SHARED((N_NODES, D_FEAT), jnp.float32),
        pltpu.VMEM((_B,), jnp.int32),
        pltpu.VMEM((_B,), jnp.int32),
        pltpu.VMEM((_B, D_FEAT), jnp.float32),
        pltpu.VMEM((_B, D_FEAT), jnp.float32),
        pltpu.VMEM((_B, D_FEAT), jnp.float32),
        pltpu.VMEM((_B, D_EDGE), jnp.float32),
        pltpu.VMEM((_BLD, D_FEAT), jnp.float32),
        pltpu.VMEM((_BLD, D_FEAT), jnp.float32),
        pltpu.VMEM((D_GLOBAL,), jnp.float32),
        pltpu.SemaphoreType.DMA,
    ],
)
def _edge_block(edges_hbm, nodes_hbm, global_hbm, recv_hbm, send_hbm, out_hbm,
                rot_sp, ridx, sidx, bufa, bufb, bufc, ebuf, nbuf, rbuf, gtmp,
                sem):
    cid = lax.axis_index("c")
    sid = lax.axis_index("s")
    wid = sid * _NC + cid

    row0 = sid * _ROWS_MAIN
    nrows = jnp.where(sid == _NS - 1, _ROWS_LAST, _ROWS_MAIN)

    @pl.loop(0, _ROWS_LAST // _BLD)
    def _build(blk):
        @pl.when(blk * _BLD < nrows)
        def _():
            r = row0 + blk * _BLD
            pltpu.sync_copy(nodes_hbm.at[pl.ds(r, _BLD)], nbuf)

            @pl.loop(0, _BLD)
            def _rot(i):
                rbuf[i, pl.ds(0, 16)] = nbuf[i, pl.ds(D_FEAT - 16, 16)]
                for k in range((D_FEAT - D_EDGE) // 16):
                    rbuf[i, pl.ds(16 + 16 * k, 16)] = nbuf[i, pl.ds(16 * k, 16)]

            pltpu.sync_copy(rbuf, rot_sp.at[pl.ds(r, _BLD)])

    pltpu.sync_copy(global_hbm, gtmp)
    gvec = gtmp[...]
    zvec = jnp.zeros_like(gvec)

    @pl.loop(0, _B)
    def _fillc(i):
        bufc[i, pl.ds(D_GLOBAL, D_GLOBAL)] = gvec
        for k in range((D_PAD - D_OUT) // 16):
            bufc[i, pl.ds(2 * D_GLOBAL + 16 * k, 16)] = zvec

    plsc.subcore_barrier()

    @pl.loop(0, _STEPS + 1)
    def _chunk(step):
        base = wid * _E_PER_W + jnp.minimum(step * _B, _E_PER_W - _B)
        rows = pl.ds(base, _B)
        pltpu.sync_copy(recv_hbm.at[rows], ridx)
        pltpu.sync_copy(send_hbm.at[rows], sidx)
        pltpu.async_copy(rot_sp.at[ridx], bufa, sem).wait()
        pltpu.async_copy(rot_sp.at[sidx], bufb, sem).wait()
        pltpu.sync_copy(edges_hbm.at[rows], ebuf)

        @pl.loop(0, _B)
        def _assemble(i):
            bufc[i, pl.ds(0, 16)] = bufb[i, pl.ds(0, 16)]
            bufb[i, pl.ds(0, 16)] = bufa[i, pl.ds(0, 16)]
            bufa[i, pl.ds(0, D_EDGE)] = ebuf[i, :]

        pltpu.sync_copy(bufa, out_hbm.at[rows, pl.ds(0, D_FEAT)])
        pltpu.sync_copy(bufb, out_hbm.at[rows, pl.ds(D_FEAT, D_FEAT)])
        pltpu.sync_copy(bufc, out_hbm.at[rows, pl.ds(2 * D_FEAT, D_FEAT)])


def kernel(edges_data, nodes_data, global_data, receivers, senders):
    padded = _edge_block(
        edges_data,
        nodes_data,
        global_data,
        receivers.astype(jnp.int32),
        senders.astype(jnp.int32),
    )
    return padded[:, :D_OUT]


# speedup vs baseline: 1.1379x; 1.1379x over previous
import functools

import jax
import jax.numpy as jnp
from jax import lax
from jax.experimental import pallas as pl
from jax.experimental.pallas import tpu as pltpu
from jax.experimental.pallas import tpu_sc as plsc

N_NODES = 10000
N_EDGES = 320000
D_FEAT = 128
D_EDGE = 16
D_GLOBAL = 16
D_OUT = D_EDGE + 2 * D_FEAT + D_GLOBAL
D_PAD = 384

_NC = 2
_NS = 16
_NW = _NC * _NS
_E_PER_W = N_EDGES // _NW
_B = 200
_STEPS = _E_PER_W // _B

_mesh = plsc.VectorSubcoreMesh(core_axis_name="c", subcore_axis_name="s")


@functools.partial(
    pl.kernel,
    out_type=jax.ShapeDtypeStruct((N_EDGES, D_PAD), jnp.float32),
    mesh=_mesh,
    scratch_types=[
        pltpu.VMEM((_B,), jnp.int32),
        pltpu.VMEM((_B,), jnp.int32),
        pltpu.VMEM((_B, D_PAD), jnp.float32),
        pltpu.VMEM((_B, D_EDGE), jnp.float32),
        pltpu.VMEM((D_GLOBAL,), jnp.float32),
        pltpu.SemaphoreType.DMA,
        pltpu.SemaphoreType.DMA,
    ],
)
def _edge_block(edges_hbm, rot_hbm, global_hbm, recv_hbm, send_hbm, out_hbm,
                ridx, sidx, obuf, ebuf, gtmp, sem, sem2):
    wid = lax.axis_index("s") * _NC + lax.axis_index("c")

    pltpu.sync_copy(global_hbm, gtmp)
    gvec = gtmp[...]
    zvec = jnp.zeros_like(gvec)

    @pl.loop(0, _B)
    def _fill(i):
        obuf[i, pl.ds(D_EDGE + 2 * D_FEAT, D_GLOBAL)] = gvec
        for k in range((D_PAD - D_OUT) // 16):
            obuf[i, pl.ds(D_OUT + 16 * k, 16)] = zvec

    @pl.loop(0, _STEPS)
    def _chunk(step):
        base = wid * _E_PER_W + step * _B
        rows = pl.ds(base, _B)
        pltpu.sync_copy(recv_hbm.at[rows], ridx)
        pltpu.sync_copy(send_hbm.at[rows], sidx)
        ca = pltpu.async_copy(
            rot_hbm.at[ridx], obuf.at[:, pl.ds(0, D_FEAT)], sem)
        cb = pltpu.async_copy(
            rot_hbm.at[sidx], obuf.at[:, pl.ds(D_FEAT, D_FEAT)], sem2)
        pltpu.sync_copy(edges_hbm.at[rows], ebuf)
        ca.wait()
        cb.wait()

        @pl.loop(0, _B)
        def _assemble(i):
            obuf[i, pl.ds(2 * D_FEAT, 16)] = obuf[i, pl.ds(D_FEAT, 16)]
            obuf[i, pl.ds(D_FEAT, 16)] = obuf[i, pl.ds(0, 16)]
            obuf[i, pl.ds(0, D_EDGE)] = ebuf[i, :]

        pltpu.sync_copy(obuf, out_hbm.at[rows, :])


def kernel(edges_data, nodes_data, global_data, receivers, senders):
    perm = jnp.equal(
        (jnp.arange(D_FEAT)[:, None] + D_EDGE) % D_FEAT,
        jnp.arange(D_FEAT)[None, :]).astype(jnp.float32)
    rot = jax.lax.dot(nodes_data, perm,
                      precision=jax.lax.Precision.HIGHEST)
    padded = _edge_block(
        edges_data,
        rot,
        global_data,
        receivers.astype(jnp.int32),
        senders.astype(jnp.int32),
    )
    return padded[:, :D_OUT]


# speedup vs baseline: 1.1704x; 1.0286x over previous
import functools

import jax
import jax.numpy as jnp
from jax import lax
from jax.experimental import pallas as pl
from jax.experimental.pallas import tpu as pltpu
from jax.experimental.pallas import tpu_sc as plsc

N_NODES = 10000
N_EDGES = 320000
D_FEAT = 128
D_EDGE = 16
D_GLOBAL = 16
D_OUT = D_EDGE + 2 * D_FEAT + D_GLOBAL
D_PAD = 384

_NC = 2
_NS = 16
_NW = _NC * _NS
_E_PER_W = N_EDGES // _NW
_B = 200
_STEPS = _E_PER_W // _B

_mesh = plsc.VectorSubcoreMesh(core_axis_name="c", subcore_axis_name="s")


@functools.partial(
    pl.kernel,
    out_type=jax.ShapeDtypeStruct((N_EDGES, D_PAD), jnp.float32),
    mesh=_mesh,
    scratch_types=[
        pltpu.VMEM((_E_PER_W,), jnp.int32),
        pltpu.VMEM((_E_PER_W,), jnp.int32),
        pltpu.VMEM((_B, D_PAD), jnp.float32),
        pltpu.VMEM((_B, D_EDGE), jnp.float32),
        pltpu.VMEM((D_GLOBAL,), jnp.float32),
        pltpu.SemaphoreType.DMA,
        pltpu.SemaphoreType.DMA,
    ],
)
def _edge_block(edges_hbm, rot_hbm, global_hbm, recv_hbm, send_hbm, out_hbm,
                ridx, sidx, obuf, ebuf, gtmp, sem, sem2):
    wid = lax.axis_index("s") * _NC + lax.axis_index("c")

    pltpu.sync_copy(global_hbm, gtmp)
    gvec = gtmp[...]
    zvec = jnp.zeros_like(gvec)

    @pl.loop(0, _B)
    def _fill(i):
        obuf[i, pl.ds(D_EDGE + 2 * D_FEAT, D_GLOBAL)] = gvec
        for k in range((D_PAD - D_OUT) // 16):
            obuf[i, pl.ds(D_OUT + 16 * k, 16)] = zvec

    wrows = pl.ds(wid * _E_PER_W, _E_PER_W)
    pltpu.sync_copy(recv_hbm.at[wrows], ridx)
    pltpu.sync_copy(send_hbm.at[wrows], sidx)

    @pl.loop(0, _STEPS)
    def _chunk(step):
        base = wid * _E_PER_W + step * _B
        rows = pl.ds(base, _B)
        off = pl.ds(step * _B, _B)
        ca = pltpu.async_copy(
            rot_hbm.at[ridx.at[off]], obuf.at[:, pl.ds(0, D_FEAT)], sem)
        cb = pltpu.async_copy(
            rot_hbm.at[sidx.at[off]], obuf.at[:, pl.ds(D_FEAT, D_FEAT)], sem2)
        pltpu.sync_copy(edges_hbm.at[rows], ebuf)
        ca.wait()
        cb.wait()

        @pl.loop(0, _B, unroll=4)
        def _assemble(i):
            obuf[i, pl.ds(2 * D_FEAT, 16)] = obuf[i, pl.ds(D_FEAT, 16)]
            obuf[i, pl.ds(D_FEAT, 16)] = obuf[i, pl.ds(0, 16)]
            obuf[i, pl.ds(0, D_EDGE)] = ebuf[i, :]

        pltpu.sync_copy(obuf, out_hbm.at[rows, :])


def kernel(edges_data, nodes_data, global_data, receivers, senders):
    perm = jnp.equal(
        (jnp.arange(D_FEAT)[:, None] + D_EDGE) % D_FEAT,
        jnp.arange(D_FEAT)[None, :]).astype(jnp.float32)
    rot = jax.lax.dot(nodes_data, perm,
                      precision=jax.lax.Precision.HIGHEST)
    padded = _edge_block(
        edges_data,
        rot,
        global_data,
        receivers.astype(jnp.int32),
        senders.astype(jnp.int32),
    )
    return padded[:, :D_OUT]


# speedup vs baseline: 1.1875x; 1.0146x over previous
import functools

import jax
import jax.numpy as jnp
from jax import lax
from jax.experimental import pallas as pl
from jax.experimental.pallas import tpu as pltpu
from jax.experimental.pallas import tpu_sc as plsc

N_NODES = 10000
N_EDGES = 320000
D_FEAT = 128
D_EDGE = 16
D_GLOBAL = 16
D_OUT = D_EDGE + 2 * D_FEAT + D_GLOBAL
D_PAD = 384

_NC = 2
_NS = 16
_NW = _NC * _NS
_E_PER_W = N_EDGES // _NW
_B = 104
_CHUNKS = _E_PER_W // _B + 1
_PAIRS = (_CHUNKS + 1) // 2

_mesh = plsc.VectorSubcoreMesh(core_axis_name="c", subcore_axis_name="s")


@functools.partial(
    pl.kernel,
    out_type=jax.ShapeDtypeStruct((N_EDGES, D_PAD), jnp.float32),
    mesh=_mesh,
    scratch_types=[
        pltpu.VMEM((_E_PER_W,), jnp.int32),
        pltpu.VMEM((_E_PER_W,), jnp.int32),
        pltpu.VMEM((_B, D_PAD), jnp.float32),
        pltpu.VMEM((_B, D_PAD), jnp.float32),
        pltpu.VMEM((_B, D_EDGE), jnp.float32),
        pltpu.VMEM((_B, D_EDGE), jnp.float32),
        pltpu.VMEM((D_GLOBAL,), jnp.float32),
        pltpu.SemaphoreType.DMA,
        pltpu.SemaphoreType.DMA,
        pltpu.SemaphoreType.DMA,
        pltpu.SemaphoreType.DMA,
    ],
)
def _edge_block(edges_hbm, rot_hbm, global_hbm, recv_hbm, send_hbm, out_hbm,
                ridx, sidx, obuf0, obuf1, ebuf0, ebuf1, gtmp,
                semr, sems, semo0, semo1):
    wid = lax.axis_index("s") * _NC + lax.axis_index("c")

    pltpu.sync_copy(global_hbm, gtmp)
    gvec = gtmp[...]
    zvec = jnp.zeros_like(gvec)

    @pl.loop(0, _B)
    def _fill(i):
        for buf in (obuf0, obuf1):
            buf[i, pl.ds(D_EDGE + 2 * D_FEAT, D_GLOBAL)] = gvec
            for k in range((D_PAD - D_OUT) // 16):
                buf[i, pl.ds(D_OUT + 16 * k, 16)] = zvec

    wrows = pl.ds(wid * _E_PER_W, _E_PER_W)
    pltpu.sync_copy(recv_hbm.at[wrows], ridx)
    pltpu.sync_copy(send_hbm.at[wrows], sidx)

    def chunk_front(step, obuf, ebuf):
        base = wid * _E_PER_W + jnp.minimum(step * _B, _E_PER_W - _B)
        rows = pl.ds(base, _B)
        off = pl.ds(jnp.minimum(step * _B, _E_PER_W - _B), _B)
        ca = pltpu.async_copy(
            rot_hbm.at[ridx.at[off]], obuf.at[:, pl.ds(0, D_FEAT)], semr)
        cb = pltpu.async_copy(
            rot_hbm.at[sidx.at[off]], obuf.at[:, pl.ds(D_FEAT, D_FEAT)], sems)
        pltpu.sync_copy(edges_hbm.at[rows], ebuf)
        return rows, ca, cb

    def chunk_back(rows, ca, cb, obuf, ebuf, semo):
        ca.wait()
        cb.wait()

        @pl.loop(0, _B, unroll=4)
        def _assemble(i):
            obuf[i, pl.ds(2 * D_FEAT, 16)] = obuf[i, pl.ds(D_FEAT, 16)]
            obuf[i, pl.ds(D_FEAT, 16)] = obuf[i, pl.ds(0, 16)]
            obuf[i, pl.ds(0, D_EDGE)] = ebuf[i, :]

        return pltpu.async_copy(obuf, out_hbm.at[rows, :], semo)

    @pl.loop(0, _PAIRS)
    def _pair(g):
        r0, ca0, cb0 = chunk_front(2 * g, obuf0, ebuf0)
        w0 = chunk_back(r0, ca0, cb0, obuf0, ebuf0, semo0)
        r1, ca1, cb1 = chunk_front(2 * g + 1, obuf1, ebuf1)
        w0.wait()
        w1 = chunk_back(r1, ca1, cb1, obuf1, ebuf1, semo1)
        w1.wait()


def kernel(edges_data, nodes_data, global_data, receivers, senders):
    perm = jnp.equal(
        (jnp.arange(D_FEAT)[:, None] + D_EDGE) % D_FEAT,
        jnp.arange(D_FEAT)[None, :]).astype(jnp.float32)
    rot = jax.lax.dot(nodes_data, perm,
                      precision=jax.lax.Precision.HIGHEST)
    padded = _edge_block(
        edges_data,
        rot,
        global_data,
        receivers.astype(jnp.int32),
        senders.astype(jnp.int32),
    )
    return padded[:, :D_OUT]
